# baseline (speedup 1.0000x reference)
import functools

import jax
import jax.numpy as jnp
from jax import lax
from jax.experimental import pallas as pl
from jax.experimental.pallas import tpu as pltpu
from jax.experimental.pallas import tpu_sc as plsc

L = 16


def _rsqrt(v):
    i = lax.bitcast_convert_type(v, jnp.int32)
    i = jnp.full((L,), 0x5F3759DF, jnp.int32) - lax.shift_right_logical(i, 1)
    y = lax.bitcast_convert_type(i, jnp.float32)
    half = v * 0.5
    for _ in range(3):
        y = y * (1.5 - half * y * y)
    return y


def _make_kernel(B, S, V, D, NC, NS):
    NW = NC * NS
    N = B * S
    CHUNK = 128
    per_w = N // NW
    n_chunks = per_w // CHUNK
    assert N % NW == 0 and per_w % CHUNK == 0 and D % L == 0
    KD = D // L

    mesh = plsc.VectorSubcoreMesh(core_axis_name="c", subcore_axis_name="s")

    @functools.partial(
        pl.kernel,
        mesh=mesh,
        compiler_params=pltpu.CompilerParams(use_tc_tiling_on_sc=False),
        out_type=jax.ShapeDtypeStruct((N, D), jnp.float32),
        scratch_types=[
            pltpu.VMEM((n_chunks, CHUNK), jnp.int32),
            pltpu.VMEM((CHUNK, D), jnp.float32),
            pltpu.VMEM((CHUNK, D), jnp.float32),
            pltpu.VMEM((S, D), jnp.float32),
            pltpu.VMEM((D,), jnp.float32),
            pltpu.VMEM((D,), jnp.float32),
            pltpu.SemaphoreType.DMA,
            pltpu.SemaphoreType.DMA,
            pltpu.SemaphoreType.DMA,
            pltpu.SemaphoreType.DMA,
        ],
    )
    def k(x_hbm, table_hbm, gamma_hbm, beta_hbm, pos_hbm, out_hbm,
          idx_v, rows0, rows1, pos_v, gam_v, bet_v,
          gsem0, gsem1, ssem0, ssem1):
        wid = lax.axis_index("s") * NC + lax.axis_index("c")
        wbase = wid * per_w

        pltpu.sync_copy(x_hbm.at[wid], idx_v)
        pltpu.sync_copy(pos_hbm, pos_v)
        pltpu.sync_copy(gamma_hbm, gam_v)
        pltpu.sync_copy(beta_hbm, bet_v)

        gvs = [gam_v[pl.ds(L * t, L)] for t in range(KD)]
        bvs = [bet_v[pl.ds(L * t, L)] for t in range(KD)]
        inv_d = jnp.float32(1.0 / D)

        lane_ids = lax.iota(jnp.int32, L)
        perms = [jnp.reshape(lane_ids ^ sh, (L, 1)) for sh in (8, 4, 2, 1)]
        _dnums = lax.GatherDimensionNumbers(
            offset_dims=(), collapsed_slice_dims=(0,), start_index_map=(0,))

        def bsum(v):
            for p in perms:
                v = v + lax.gather(
                    v, p, _dnums, slice_sizes=(1,), unique_indices=True,
                    mode=lax.GatherScatterMode.PROMISE_IN_BOUNDS)
            return v

        def start_gather(c, rows, gsem):
            pltpu.async_copy(table_hbm.at[idx_v.at[c]], rows, gsem)

        def wait_gather(rows, gsem):
            pltpu.make_async_copy(table_hbm.at[pl.ds(0, CHUNK)], rows, gsem).wait()

        def wait_store(c, rows, ssem):
            pltpu.make_async_copy(rows, out_hbm.at[pl.ds(c * CHUNK, CHUNK)],
                                  ssem).wait()

        start_gather(0, rows0, gsem0)

        def do_chunk(c, rows, gsem, ssem, n_rows, n_gsem, n_ssem):
            base = wbase + c * CHUNK
            wait_gather(rows, gsem)
            @pl.when(c + 1 < n_chunks)
            def _():
                @pl.when(c >= 1)
                def _():
                    wait_store(c - 1, n_rows, n_ssem)
                start_gather(c + 1, n_rows, n_gsem)

            p0 = lax.rem(base, S)

            def row_body(j, _):
                p = p0 + j
                p = jnp.where(p < S, p, p - S)
                hs = [rows[j, pl.ds(L * t, L)] + pos_v[p, pl.ds(L * t, L)]
                      for t in range(KD)]
                tot = bsum((hs[0] + hs[1]) + (hs[2] + hs[3]))
                sq = ((hs[0] * hs[0] + hs[1] * hs[1])
                      + (hs[2] * hs[2] + hs[3] * hs[3]))
                tot2 = bsum(sq)
                mean_v = tot * inv_d
                var = tot2 * inv_d - mean_v * mean_v + 1e-5
                inv = _rsqrt(var)
                for t in range(KD):
                    rows[j, pl.ds(L * t, L)] = ((hs[t] - mean_v) * inv
                                                * gvs[t] + bvs[t])
                return 0

            lax.fori_loop(0, CHUNK, row_body, 0, unroll=2)
            pltpu.async_copy(rows, out_hbm.at[pl.ds(base, CHUNK)], ssem)

        def outer(go, _):
            for b in range(2):
                c = go * 2 + b
                if b == 0:
                    do_chunk(c, rows0, gsem0, ssem0, rows1, gsem1, ssem1)
                else:
                    do_chunk(c, rows1, gsem1, ssem1, rows0, gsem0, ssem0)
            return 0

        lax.fori_loop(0, n_chunks // 2, outer, 0)
        wait_store(n_chunks - 2, rows0, ssem0)
        wait_store(n_chunks - 1, rows1, ssem1)

    return k


def kernel(x, tok_table, gamma, beta, pos_embed):
    B, S = x.shape
    V, D = tok_table.shape
    info = plsc.get_sparse_core_info()
    NC, NS = info.num_cores, info.num_subcores
    NW = NC * NS
    N = B * S
    CHUNK = 128
    per_w = N // NW
    k = _make_kernel(B, S, V, D, NC, NS)
    x_resh = x.reshape(NW, per_w // CHUNK, CHUNK)
    out = k(x_resh, tok_table, gamma, beta, pos_embed)
    return out.reshape(B, S, D)


# speedup vs baseline: 1.0458x; 1.0458x over previous
import functools

import jax
import jax.numpy as jnp
from jax import lax
from jax.experimental import pallas as pl
from jax.experimental.pallas import tpu as pltpu
from jax.experimental.pallas import tpu_sc as plsc

L = 16


def _rsqrt(v):
    i = lax.bitcast_convert_type(v, jnp.int32)
    i = jnp.full((L,), 0x5F3759DF, jnp.int32) - lax.shift_right_logical(i, 1)
    y = lax.bitcast_convert_type(i, jnp.float32)
    half = v * 0.5
    for _ in range(2):
        y = y * (1.5 - half * y * y)
    return y


def _make_kernel(B, S, V, D, NC, NS):
    NW = NC * NS
    N = B * S
    CHUNK = 128
    per_w = N // NW
    n_chunks = per_w // CHUNK
    assert N % NW == 0 and per_w % CHUNK == 0 and D % L == 0
    KD = D // L

    mesh = plsc.VectorSubcoreMesh(core_axis_name="c", subcore_axis_name="s")

    @functools.partial(
        pl.kernel,
        mesh=mesh,
        compiler_params=pltpu.CompilerParams(use_tc_tiling_on_sc=False),
        out_type=jax.ShapeDtypeStruct((N, D), jnp.float32),
        scratch_types=[
            pltpu.VMEM((n_chunks, CHUNK), jnp.int32),
            pltpu.VMEM((CHUNK, D), jnp.float32),
            pltpu.VMEM((CHUNK, D), jnp.float32),
            pltpu.VMEM((S, D), jnp.float32),
            pltpu.VMEM((D,), jnp.float32),
            pltpu.VMEM((D,), jnp.float32),
            pltpu.SemaphoreType.DMA,
            pltpu.SemaphoreType.DMA,
            pltpu.SemaphoreType.DMA,
            pltpu.SemaphoreType.DMA,
        ],
    )
    def k(x_hbm, table_hbm, gamma_hbm, beta_hbm, pos_hbm, out_hbm,
          idx_v, rows0, rows1, pos_v, gam_v, bet_v,
          gsem0, gsem1, ssem0, ssem1):
        wid = lax.axis_index("s") * NC + lax.axis_index("c")
        wbase = wid * per_w

        pltpu.sync_copy(x_hbm.at[wid], idx_v)
        pltpu.sync_copy(pos_hbm, pos_v)
        pltpu.sync_copy(gamma_hbm, gam_v)
        pltpu.sync_copy(beta_hbm, bet_v)

        gvs = [gam_v[pl.ds(L * t, L)] for t in range(KD)]
        bvs = [bet_v[pl.ds(L * t, L)] for t in range(KD)]
        inv_d = jnp.float32(1.0 / D)

        lane_ids = lax.iota(jnp.int32, L)
        perms = [jnp.reshape(lane_ids ^ sh, (L, 1)) for sh in (8, 4, 2, 1)]
        _dnums = lax.GatherDimensionNumbers(
            offset_dims=(), collapsed_slice_dims=(0,), start_index_map=(0,))

        def bsum(v):
            for p in perms:
                v = v + lax.gather(
                    v, p, _dnums, slice_sizes=(1,), unique_indices=True,
                    mode=lax.GatherScatterMode.PROMISE_IN_BOUNDS)
            return v

        def start_gather(c, rows, gsem):
            pltpu.async_copy(table_hbm.at[idx_v.at[c]], rows, gsem)

        def wait_gather(rows, gsem):
            pltpu.make_async_copy(table_hbm.at[pl.ds(0, CHUNK)], rows, gsem).wait()

        def wait_store(c, rows, ssem):
            pltpu.make_async_copy(rows, out_hbm.at[pl.ds(c * CHUNK, CHUNK)],
                                  ssem).wait()

        start_gather(0, rows0, gsem0)

        def do_chunk(c, rows, gsem, ssem, n_rows, n_gsem, n_ssem):
            base = wbase + c * CHUNK
            wait_gather(rows, gsem)
            @pl.when(c + 1 < n_chunks)
            def _():
                @pl.when(c >= 1)
                def _():
                    wait_store(c - 1, n_rows, n_ssem)
                start_gather(c + 1, n_rows, n_gsem)

            p0 = lax.rem(base, S)

            def row_body(j, _):
                p = p0 + j
                p = jnp.where(p < S, p, p - S)
                hs = [rows[j, pl.ds(L * t, L)] + pos_v[p, pl.ds(L * t, L)]
                      for t in range(KD)]
                tot = bsum((hs[0] + hs[1]) + (hs[2] + hs[3]))
                sq = ((hs[0] * hs[0] + hs[1] * hs[1])
                      + (hs[2] * hs[2] + hs[3] * hs[3]))
                tot2 = bsum(sq)
                mean_v = tot * inv_d
                var = tot2 * inv_d - mean_v * mean_v + 1e-5
                inv = _rsqrt(var)
                for t in range(KD):
                    rows[j, pl.ds(L * t, L)] = ((hs[t] - mean_v) * inv
                                                * gvs[t] + bvs[t])
                return 0

            lax.fori_loop(0, CHUNK, row_body, 0, unroll=8)
            pltpu.async_copy(rows, out_hbm.at[pl.ds(base, CHUNK)], ssem)

        def outer(go, _):
            for b in range(2):
                c = go * 2 + b
                if b == 0:
                    do_chunk(c, rows0, gsem0, ssem0, rows1, gsem1, ssem1)
                else:
                    do_chunk(c, rows1, gsem1, ssem1, rows0, gsem0, ssem0)
            return 0

        lax.fori_loop(0, n_chunks // 2, outer, 0)
        wait_store(n_chunks - 2, rows0, ssem0)
        wait_store(n_chunks - 1, rows1, ssem1)

    return k


def kernel(x, tok_table, gamma, beta, pos_embed):
    B, S = x.shape
    V, D = tok_table.shape
    info = plsc.get_sparse_core_info()
    NC, NS = info.num_cores, info.num_subcores
    NW = NC * NS
    N = B * S
    CHUNK = 128
    per_w = N // NW
    k = _make_kernel(B, S, V, D, NC, NS)
    x_resh = x.reshape(NW, per_w // CHUNK, CHUNK)
    out = k(x_resh, tok_table, gamma, beta, pos_embed)
    return out.reshape(B, S, D)


# speedup vs baseline: 1.4559x; 1.3921x over previous
import functools

import jax
import jax.numpy as jnp
from jax import lax
from jax.experimental import pallas as pl
from jax.experimental.pallas import tpu as pltpu
from jax.experimental.pallas import tpu_sc as plsc

L = 16


def _rsqrt(v):
    i = lax.bitcast_convert_type(v, jnp.int32)
    i = jnp.full((L,), 0x5F3759DF, jnp.int32) - lax.shift_right_logical(i, 1)
    y = lax.bitcast_convert_type(i, jnp.float32)
    half = v * 0.5
    for _ in range(2):
        y = y * (1.5 - half * y * y)
    return y


def _make_kernel(B, S, V, D, NC, NS):
    NW = NC * NS
    N = B * S
    CHUNK = 128
    per_w = N // NW
    n_chunks = per_w // CHUNK
    assert N % NW == 0 and per_w % CHUNK == 0 and D % L == 0
    KD = D // L

    mesh = plsc.VectorSubcoreMesh(core_axis_name="c", subcore_axis_name="s")

    @functools.partial(
        pl.kernel,
        mesh=mesh,
        compiler_params=pltpu.CompilerParams(use_tc_tiling_on_sc=False),
        out_type=jax.ShapeDtypeStruct((N, D), jnp.float32),
        scratch_types=[
            pltpu.VMEM((n_chunks, CHUNK), jnp.int32),
            pltpu.VMEM((CHUNK, D), jnp.float32),
            pltpu.VMEM((CHUNK, D), jnp.float32),
            pltpu.VMEM((S, D), jnp.float32),
            pltpu.VMEM((D,), jnp.float32),
            pltpu.VMEM((D,), jnp.float32),
            pltpu.SemaphoreType.DMA,
            pltpu.SemaphoreType.DMA,
            pltpu.SemaphoreType.DMA,
            pltpu.SemaphoreType.DMA,
        ],
    )
    def k(x_hbm, table_hbm, gamma_hbm, beta_hbm, pos_hbm, out_hbm,
          idx_v, rows0, rows1, pos_v, gam_v, bet_v,
          gsem0, gsem1, ssem0, ssem1):
        wid = lax.axis_index("s") * NC + lax.axis_index("c")
        wbase = wid * per_w

        pltpu.sync_copy(x_hbm.at[wid], idx_v)
        pltpu.sync_copy(pos_hbm, pos_v)
        pltpu.sync_copy(gamma_hbm, gam_v)
        pltpu.sync_copy(beta_hbm, bet_v)

        gvs = [gam_v[pl.ds(L * t, L)] for t in range(KD)]
        bvs = [bet_v[pl.ds(L * t, L)] for t in range(KD)]
        inv_d = jnp.float32(1.0 / D)

        lane_ids = lax.iota(jnp.int32, L)
        _dnums = lax.GatherDimensionNumbers(
            offset_dims=(), collapsed_slice_dims=(0,), start_index_map=(0,))

        def P(v, idx):
            return lax.gather(v, idx, _dnums, slice_sizes=(1,),
                              unique_indices=True,
                              mode=lax.GatherScatterMode.PROMISE_IN_BOUNDS)

        perm8i, perm4i, perm2i, perm1i = (
            jnp.reshape(lane_ids ^ sh, (L, 1)) for sh in (8, 4, 2, 1))
        m8 = lane_ids < 8
        m4 = (lane_ids & 4) == 0
        lo2 = lane_ids & 3
        bidx = [jnp.reshape(lo2 + off, (L, 1)) for off in (0, 8, 4, 12)]

        def pack4(x0, x1, x2, x3):
            t0, t1, t2, t3 = (x + P(x, perm8i) for x in (x0, x1, x2, x3))
            u01 = jnp.where(m8, t0, t1)
            u23 = jnp.where(m8, t2, t3)
            v01 = u01 + P(u01, perm4i)
            v23 = u23 + P(u23, perm4i)
            w = jnp.where(m4, v01, v23)
            w = w + P(w, perm2i)
            return w + P(w, perm1i)

        def start_gather(c, rows, gsem):
            pltpu.async_copy(table_hbm.at[idx_v.at[c]], rows, gsem)

        def wait_gather(rows, gsem):
            pltpu.make_async_copy(table_hbm.at[pl.ds(0, CHUNK)], rows, gsem).wait()

        def wait_store(c, rows, ssem):
            pltpu.make_async_copy(rows, out_hbm.at[pl.ds(c * CHUNK, CHUNK)],
                                  ssem).wait()

        start_gather(0, rows0, gsem0)

        def do_chunk(c, rows, gsem, ssem, n_rows, n_gsem, n_ssem):
            base = wbase + c * CHUNK
            wait_gather(rows, gsem)
            @pl.when(c + 1 < n_chunks)
            def _():
                @pl.when(c >= 1)
                def _():
                    wait_store(c - 1, n_rows, n_ssem)
                start_gather(c + 1, n_rows, n_gsem)

            p0 = lax.rem(base, S)

            def blk_body(jj, _):
                j = jj * 4
                pj = p0 + j
                hs = []
                ss = []
                qs = []
                for r in range(4):
                    pr = pj + r
                    pr = jnp.where(pr < S, pr, pr - S)
                    h = [rows[j + r, pl.ds(L * t, L)]
                         + pos_v[pr, pl.ds(L * t, L)] for t in range(KD)]
                    hs.append(h)
                    ss.append((h[0] + h[1]) + (h[2] + h[3]))
                    qs.append((h[0] * h[0] + h[1] * h[1])
                              + (h[2] * h[2] + h[3] * h[3]))
                s4 = pack4(*ss)
                q4 = pack4(*qs)
                mean4 = s4 * inv_d
                var4 = q4 * inv_d - mean4 * mean4 + 1e-5
                inv4 = _rsqrt(var4)
                for r in range(4):
                    mean_r = P(mean4, bidx[r])
                    inv_r = P(inv4, bidx[r])
                    for t in range(KD):
                        rows[j + r, pl.ds(L * t, L)] = (
                            (hs[r][t] - mean_r) * inv_r * gvs[t] + bvs[t])
                return 0

            lax.fori_loop(0, CHUNK // 4, blk_body, 0, unroll=2)
            pltpu.async_copy(rows, out_hbm.at[pl.ds(base, CHUNK)], ssem)

        def outer(go, _):
            for b in range(2):
                c = go * 2 + b
                if b == 0:
                    do_chunk(c, rows0, gsem0, ssem0, rows1, gsem1, ssem1)
                else:
                    do_chunk(c, rows1, gsem1, ssem1, rows0, gsem0, ssem0)
            return 0

        lax.fori_loop(0, n_chunks // 2, outer, 0)
        wait_store(n_chunks - 2, rows0, ssem0)
        wait_store(n_chunks - 1, rows1, ssem1)

    return k


def kernel(x, tok_table, gamma, beta, pos_embed):
    B, S = x.shape
    V, D = tok_table.shape
    info = plsc.get_sparse_core_info()
    NC, NS = info.num_cores, info.num_subcores
    NW = NC * NS
    N = B * S
    CHUNK = 128
    per_w = N // NW
    k = _make_kernel(B, S, V, D, NC, NS)
    x_resh = x.reshape(NW, per_w // CHUNK, CHUNK)
    out = k(x_resh, tok_table, gamma, beta, pos_embed)
    return out.reshape(B, S, D)
